# baseline (speedup 1.0000x reference)
import functools

import jax
import jax.numpy as jnp
from jax import lax
from jax.experimental import pallas as pl
from jax.experimental.pallas import tpu as pltpu
from jax.experimental.pallas import tpu_sc as plsc

N = 10000
E = 320000
G = 64
D = 64

NC = 2
NS = 16
NW = NC * NS

EPT = E // NW
CHUNK = 80
CH_PER_TILE = 125
ROWS_SP = 10240

NB = 5
DA = 3
NSTEP = CH_PER_TILE


def _make_seg_sum():
  mesh = plsc.VectorSubcoreMesh(core_axis_name="c", subcore_axis_name="s")

  @functools.partial(
      pl.kernel,
      mesh=mesh,
      compiler_params=pltpu.CompilerParams(use_tc_tiling_on_sc=False),
      out_type=jax.ShapeDtypeStruct((NC, N, D), jnp.float32),
      scratch_types=[
          pltpu.VMEM((EPT,), jnp.int32),
          pltpu.VMEM((EPT,), jnp.int32),
          pltpu.VMEM((NB, CHUNK, D), jnp.float32),
          pltpu.VMEM_SHARED((N, D), jnp.float32),
          pltpu.VMEM_SHARED((ROWS_SP, D), jnp.float32),
          pltpu.SemaphoreType.DMA((NB,)),
          pltpu.SemaphoreType.DMA((NB,)),
      ],
  )
  def seg_sum(h_hbm, eidx_hbm, zeros_hbm, out_hbm,
              src1d, dst1d, rows, h_sh, acc_sh, gsem, ssem):
    c = lax.axis_index("c")
    s = lax.axis_index("s")
    wid = c * NS + s

    hrows = 624
    hbase = s * hrows
    zrows = ROWS_SP // NS
    zbase = s * zrows
    c1 = pltpu.async_copy(eidx_hbm.at[0].at[pl.ds(wid * EPT, EPT)], src1d,
                          gsem.at[0])
    c2 = pltpu.async_copy(eidx_hbm.at[1].at[pl.ds(wid * EPT, EPT)], dst1d,
                          gsem.at[1])
    c3 = pltpu.async_copy(h_hbm.at[pl.ds(hbase, hrows)],
                          h_sh.at[pl.ds(hbase, hrows)], gsem.at[2])
    c4 = pltpu.async_copy(zeros_hbm.at[pl.ds(zbase, zrows)],
                          acc_sh.at[pl.ds(zbase, zrows)], gsem.at[3])

    @pl.when(s == 0)
    def _h_tail():
      pltpu.async_copy(h_hbm.at[pl.ds(NS * hrows, N - NS * hrows)],
                       h_sh.at[pl.ds(NS * hrows, N - NS * hrows)],
                       gsem.at[4]).wait()

    c1.wait()
    c2.wait()
    c3.wait()
    c4.wait()
    plsc.subcore_barrier()

    def sidx(k):
      return src1d.at[pl.ds(k * CHUNK, CHUNK)]

    def didx(k):
      return dst1d.at[pl.ds(k * CHUNK, CHUNK)]

    def gather(k, b):
      return pltpu.make_async_copy(h_sh.at[sidx(k)], rows.at[b], gsem.at[b])

    def scatter(k, b):
      return pltpu.make_async_copy(rows.at[b], acc_sh.at[didx(k)],
                                   ssem.at[b])

    for b in range(DA):
      pltpu.async_copy(h_sh.at[sidx(b)], rows.at[b], gsem.at[b])

    def body(jj, carry):
      base = jj * NB
      for b in range(NB):
        k = base + b
        gather(k, b).wait()
        pltpu.async_copy(rows.at[b], acc_sh.at[didx(k)], ssem.at[b],
                         add=True)
        kn = k + DA
        bn = (b + DA) % NB

        @pl.when(jnp.logical_and(kn >= NB, kn < NSTEP))
        def _wait_prev_scatter():
          scatter(kn - NB, bn).wait()

        @pl.when(kn < NSTEP)
        def _fire_ahead():
          pltpu.async_copy(h_sh.at[sidx(kn)], rows.at[bn], gsem.at[bn])
      return carry

    lax.fori_loop(0, NSTEP // NB, body, 0)
    for b in range(NB):
      scatter(NSTEP - NB + b, b).wait()
    plsc.subcore_barrier()

    orows = 624
    obase = s * orows
    pltpu.sync_copy(acc_sh.at[pl.ds(obase, orows)],
                    out_hbm.at[c].at[pl.ds(obase, orows)])

    @pl.when(s == 0)
    def _tail():
      pltpu.sync_copy(acc_sh.at[pl.ds(NS * orows, N - NS * orows)],
                      out_hbm.at[c].at[pl.ds(NS * orows, N - NS * orows)])

  return seg_sum


_SEG = _make_seg_sum()


N2 = N // 2


def _dup(v):
  return jnp.concatenate([v, v], axis=1)


def _mm2(z, W, half):
  return jnp.concatenate(
      [jnp.dot(z[:, :half], W, preferred_element_type=jnp.float32),
       jnp.dot(z[:, half:], W, preferred_element_type=jnp.float32)], axis=1)


def _bn_packed(z, g, b, half):
  m = jnp.mean(z, axis=0, keepdims=True)
  q = jnp.mean(z * z, axis=0, keepdims=True)
  mf = (m[:, :half] + m[:, half:]) * 0.5
  qf = (q[:, :half] + q[:, half:]) * 0.5
  var = jnp.maximum(qf - mf * mf, 0.0)
  scale = _dup(g * lax.rsqrt(var + 1e-5))
  shift = _dup(b - g * mf * lax.rsqrt(var + 1e-5))
  return z * scale + shift


def _pre(x, W1):
  dh = W1.shape[1]

  def body(x_ref, w_ref, out_ref):
    out_ref[...] = jnp.dot(x_ref[...], w_ref[...],
                           preferred_element_type=jnp.float32)

  return pl.pallas_call(
      body, out_shape=jax.ShapeDtypeStruct((N, dh), jnp.float32))(x, W1)


def _dense_layer(u2, agg2, b1, g1, be1, W2, b2, gm, bm, W1n):
  has_next = W1n is not None

  def body(*refs):
    (u_ref, agg_ref, b1_ref, g1_ref, be1_ref, w2_ref, b2_ref,
     gm_ref, bm_ref) = refs[:9]
    out_ref = refs[-1]
    z = u_ref[...] + agg_ref[0] + agg_ref[1] + _dup(b1_ref[...])
    z = _bn_packed(z, g1_ref[...], be1_ref[...], D)
    z = jnp.maximum(z, 0.0)
    z = _mm2(z, w2_ref[...], D) + _dup(b2_ref[...])
    z = _bn_packed(z, gm_ref[...], bm_ref[...], D)
    z = jnp.maximum(z, 0.0)
    if has_next:
      z = _mm2(z, refs[9][...], D)
    out_ref[...] = z

  args = [u2, agg2, b1.reshape(1, -1), g1.reshape(1, -1),
          be1.reshape(1, -1), W2, b2.reshape(1, -1),
          gm.reshape(1, -1), bm.reshape(1, -1)]
  if has_next:
    args.append(W1n)
  return pl.pallas_call(
      body,
      out_shape=jax.ShapeDtypeStruct((N2, 2 * D), jnp.float32),
  )(*args)


def _final_layer(h2, agg2, W1, b1, g1, be1, W2, b2, bt_e, bt_o):
  dout = W2.shape[1]

  def body(h_ref, agg_ref, w1_ref, b1_ref, g1_ref, be1_ref,
           w2_ref, b2_ref, bte_ref, bto_ref, out_ref):
    z = h_ref[...] + agg_ref[0] + agg_ref[1]
    z = _mm2(z, w1_ref[...], D) + _dup(b1_ref[...])
    z = _bn_packed(z, g1_ref[...], be1_ref[...], dout)
    z = jnp.maximum(z, 0.0)
    z = _mm2(z, w2_ref[...], dout) + _dup(b2_ref[...])
    oh_e = (lax.broadcasted_iota(jnp.int32, (G, N2), 0) ==
            bte_ref[...]).astype(jnp.float32)
    oh_o = (lax.broadcasted_iota(jnp.int32, (G, N2), 0) ==
            bto_ref[...]).astype(jnp.float32)
    se = jnp.dot(oh_e, z, preferred_element_type=jnp.float32)
    so = jnp.dot(oh_o, z, preferred_element_type=jnp.float32)
    sums = se[:, :dout] + so[:, dout:]
    cnt = jnp.sum(oh_e + oh_o, axis=1, keepdims=True)
    mean = sums / jnp.maximum(cnt, 1.0)
    mx = jnp.max(mean, axis=1, keepdims=True)
    lse = jnp.log(jnp.sum(jnp.exp(mean - mx), axis=1, keepdims=True)) + mx
    out_ref[...] = mean - lse

  return pl.pallas_call(
      body,
      out_shape=jax.ShapeDtypeStruct((G, dout), jnp.float32),
  )(h2, agg2, W1, b1.reshape(1, -1), g1.reshape(1, -1), be1.reshape(1, -1),
    W2, b2.reshape(1, -1), bt_e, bt_o)


def kernel(x, edge_index, batch, params):
  p = list(params)
  layer_p = [p[i * 6:(i + 1) * 6] for i in range(5)]
  norm_p = [p[30 + i * 2:30 + (i + 1) * 2] for i in range(4)]

  zeros = jnp.zeros((ROWS_SP, D), jnp.float32)

  u2 = _pre(x, layer_p[0][0]).reshape(N2, 2 * D)
  for i in range(4):
    _, b1, g1, be1, W2, b2 = layer_p[i]
    gm, bm = norm_p[i]
    agg = _SEG(u2.reshape(N, D), edge_index, zeros)
    agg2 = agg.reshape(2, N2, 2 * D)
    W1n = layer_p[i + 1][0] if i < 3 else None
    u2 = _dense_layer(u2, agg2, b1, g1, be1, W2, b2, gm, bm, W1n)

  W1, b1, g1, be1, W2, b2 = layer_p[4]
  agg = _SEG(u2.reshape(N, D), edge_index, zeros)
  agg2 = agg.reshape(2, N2, 2 * D)
  bt = batch.astype(jnp.int32)
  return _final_layer(u2, agg2, W1, b1, g1, be1, W2, b2,
                      bt[0::2].reshape(1, N2), bt[1::2].reshape(1, N2))
